# Initial kernel scaffold; baseline (speedup 1.0000x reference)
import jax
import jax.numpy as jnp
from jax.experimental import pallas as pl


def kernel(x, Wg, bg, We, be):
    raise NotImplementedError("write your pallas kernel here")


# speedup vs baseline: 6.1481x; 6.1481x over previous
import jax
import jax.numpy as jnp
from jax.experimental import pallas as pl


def _route_kernel(x_ref, wg_ref, bg_ref, s_ref, cw_ref):
    x = x_ref[...]
    logits = jax.lax.dot_general(
        x, wg_ref[...], (((1,), (1,)), ((), ())),
        preferred_element_type=jnp.float32) + bg_ref[...]
    E = logits.shape[1]
    eids = jax.lax.broadcasted_iota(jnp.int32, logits.shape, 1)
    v1 = jnp.max(logits, axis=1, keepdims=True)
    i1 = jnp.min(jnp.where(logits == v1, eids, E), axis=1, keepdims=True)
    oh1 = eids == i1
    masked = jnp.where(oh1, -jnp.inf, logits)
    v2 = jnp.max(masked, axis=1, keepdims=True)
    i2 = jnp.min(jnp.where(masked == v2, eids, E), axis=1, keepdims=True)
    oh2 = eids == i2
    t = jnp.exp(v2 - v1)
    w1 = 1.0 / (1.0 + t)
    w2 = t / (1.0 + t)
    coef = w1 * oh1.astype(jnp.float32) + w2 * oh2.astype(jnp.float32)
    s_ref[...] = jax.lax.dot_general(
        coef, x, (((0,), (0,)), ((), ())),
        preferred_element_type=jnp.float32)
    cw_ref[...] = jnp.sum(coef, axis=0, keepdims=True)


def _expert_kernel(s_ref, cw_ref, be_ref, we_ref, out_ref):
    e = pl.program_id(0)
    s_e = s_ref[pl.ds(e, 1), :]
    contrib = jax.lax.dot_general(
        s_e, we_ref[0], (((1,), (1,)), ((), ())),
        preferred_element_type=jnp.float32)

    @pl.when(e == 0)
    def _():
        bias = jax.lax.dot_general(
            cw_ref[...], be_ref[...], (((1,), (0,)), ((), ())),
            preferred_element_type=jnp.float32)
        out_ref[...] = contrib + bias

    @pl.when(e != 0)
    def _():
        out_ref[...] = out_ref[...] + contrib


def kernel(x, Wg, bg, We, be):
    B, D = x.shape
    E, O, _ = We.shape
    s, cw = pl.pallas_call(
        _route_kernel,
        out_shape=(jax.ShapeDtypeStruct((E, D), jnp.float32),
                   jax.ShapeDtypeStruct((1, E), jnp.float32)),
    )(x, Wg, bg.reshape(1, E))
    total = pl.pallas_call(
        _expert_kernel,
        grid=(E,),
        in_specs=[
            pl.BlockSpec((E, D), lambda e: (0, 0)),
            pl.BlockSpec((1, E), lambda e: (0, 0)),
            pl.BlockSpec((E, O), lambda e: (0, 0)),
            pl.BlockSpec((1, O, D), lambda e: (e, 0, 0)),
        ],
        out_specs=pl.BlockSpec((1, O), lambda e: (0, 0)),
        out_shape=jax.ShapeDtypeStruct((1, O), jnp.float32),
    )(s, cw, be, We)
    return jnp.broadcast_to(total, (B, O)).astype(x.dtype)


# speedup vs baseline: 6.4908x; 1.0557x over previous
import jax
import jax.numpy as jnp
from jax.experimental import pallas as pl
from jax.experimental.pallas import tpu as pltpu


def _moe_kernel(x_ref, wg_ref, bg_ref, be_ref, we_ref, out_ref, s_ref, cw_ref):
    e = pl.program_id(0)

    @pl.when(e == 0)
    def _():
        x = x_ref[...]
        logits = jax.lax.dot_general(
            x, wg_ref[...], (((1,), (1,)), ((), ())),
            preferred_element_type=jnp.float32) + bg_ref[...]
        E = logits.shape[1]
        eids = jax.lax.broadcasted_iota(jnp.int32, logits.shape, 1)
        v1 = jnp.max(logits, axis=1, keepdims=True)
        i1 = jnp.min(jnp.where(logits == v1, eids, E), axis=1, keepdims=True)
        oh1 = eids == i1
        masked = jnp.where(oh1, -jnp.inf, logits)
        v2 = jnp.max(masked, axis=1, keepdims=True)
        i2 = jnp.min(jnp.where(masked == v2, eids, E), axis=1, keepdims=True)
        oh2 = eids == i2
        t = jnp.exp(v2 - v1)
        w1 = 1.0 / (1.0 + t)
        w2 = t / (1.0 + t)
        coef = w1 * oh1.astype(jnp.float32) + w2 * oh2.astype(jnp.float32)
        s_ref[...] = jax.lax.dot_general(
            coef, x, (((0,), (0,)), ((), ())),
            preferred_element_type=jnp.float32)
        cw_ref[...] = jnp.sum(coef, axis=0, keepdims=True)

    s_e = s_ref[pl.ds(e, 1), :]
    contrib = jax.lax.dot_general(
        s_e, we_ref[0], (((1,), (1,)), ((), ())),
        preferred_element_type=jnp.float32)

    @pl.when(e == 0)
    def _():
        bias = jax.lax.dot_general(
            cw_ref[...], be_ref[...], (((1,), (0,)), ((), ())),
            preferred_element_type=jnp.float32)
        out_ref[...] = contrib + bias

    @pl.when(e != 0)
    def _():
        out_ref[...] = out_ref[...] + contrib


def kernel(x, Wg, bg, We, be):
    B, D = x.shape
    E, O, _ = We.shape
    total = pl.pallas_call(
        _moe_kernel,
        grid=(E,),
        in_specs=[
            pl.BlockSpec((B, D), lambda e: (0, 0)),
            pl.BlockSpec((E, D), lambda e: (0, 0)),
            pl.BlockSpec((1, E), lambda e: (0, 0)),
            pl.BlockSpec((E, O), lambda e: (0, 0)),
            pl.BlockSpec((1, O, D), lambda e: (e, 0, 0)),
        ],
        out_specs=pl.BlockSpec((1, O), lambda e: (0, 0)),
        out_shape=jax.ShapeDtypeStruct((1, O), jnp.float32),
        scratch_shapes=[
            pltpu.VMEM((E, D), jnp.float32),
            pltpu.VMEM((1, E), jnp.float32),
        ],
    )(x, Wg, bg.reshape(1, E), be, We)
    return jnp.broadcast_to(total, (B, O)).astype(x.dtype)


# speedup vs baseline: 6.6184x; 1.0197x over previous
import functools

import jax
import jax.numpy as jnp
from jax.experimental import pallas as pl
from jax.experimental.pallas import tpu as pltpu

_NSPLIT = 2


def _moe_kernel(nsplit, x_ref, wg_ref, bg_ref, be_ref, *rest):
    we_refs = rest[:nsplit]
    out_ref = rest[nsplit]
    s_ref, cw_ref = rest[nsplit + 1:]
    e = pl.program_id(0)

    @pl.when(e == 0)
    def _():
        x = x_ref[...]
        logits = jax.lax.dot_general(
            x, wg_ref[...], (((1,), (1,)), ((), ())),
            preferred_element_type=jnp.float32) + bg_ref[...]
        E = logits.shape[1]
        eids = jax.lax.broadcasted_iota(jnp.int32, logits.shape, 1)
        v1 = jnp.max(logits, axis=1, keepdims=True)
        i1 = jnp.min(jnp.where(logits == v1, eids, E), axis=1, keepdims=True)
        oh1 = eids == i1
        masked = jnp.where(oh1, -jnp.inf, logits)
        v2 = jnp.max(masked, axis=1, keepdims=True)
        i2 = jnp.min(jnp.where(masked == v2, eids, E), axis=1, keepdims=True)
        oh2 = eids == i2
        t = jnp.exp(v2 - v1)
        w1 = 1.0 / (1.0 + t)
        w2 = t / (1.0 + t)
        coef = w1 * oh1.astype(jnp.float32) + w2 * oh2.astype(jnp.float32)
        s_ref[...] = jax.lax.dot_general(
            coef, x, (((0,), (0,)), ((), ())),
            preferred_element_type=jnp.float32)
        cw_ref[...] = jnp.sum(coef, axis=0, keepdims=True)

    contrib = jax.lax.dot_general(
        s_ref[pl.ds(e * nsplit, 1), :], we_refs[0][0],
        (((1,), (1,)), ((), ())),
        preferred_element_type=jnp.float32)
    for j in range(1, nsplit):
        contrib = contrib + jax.lax.dot_general(
            s_ref[pl.ds(e * nsplit + j, 1), :], we_refs[j][0],
            (((1,), (1,)), ((), ())),
            preferred_element_type=jnp.float32)

    @pl.when(e == 0)
    def _():
        bias = jax.lax.dot_general(
            cw_ref[...], be_ref[...], (((1,), (0,)), ((), ())),
            preferred_element_type=jnp.float32)
        out_ref[...] = contrib + bias

    @pl.when(e != 0)
    def _():
        out_ref[...] = out_ref[...] + contrib


def kernel(x, Wg, bg, We, be):
    B, D = x.shape
    E, O, _ = We.shape
    ns = _NSPLIT
    we_specs = [
        pl.BlockSpec((1, O, D), functools.partial(
            lambda e, j: (e * ns + j, 0, 0), j=j))
        for j in range(ns)
    ]
    total = pl.pallas_call(
        functools.partial(_moe_kernel, ns),
        grid=(E // ns,),
        in_specs=[
            pl.BlockSpec((B, D), lambda e: (0, 0)),
            pl.BlockSpec((E, D), lambda e: (0, 0)),
            pl.BlockSpec((1, E), lambda e: (0, 0)),
            pl.BlockSpec((E, O), lambda e: (0, 0)),
        ] + we_specs,
        out_specs=pl.BlockSpec((1, O), lambda e: (0, 0)),
        out_shape=jax.ShapeDtypeStruct((1, O), jnp.float32),
        scratch_shapes=[
            pltpu.VMEM((E, D), jnp.float32),
            pltpu.VMEM((1, E), jnp.float32),
        ],
    )(x, Wg, bg.reshape(1, E), be, *([We] * ns))
    return jnp.broadcast_to(total, (B, O)).astype(x.dtype)
